# baseline (speedup 1.0000x reference)
import jax
import jax.numpy as jnp
from jax.experimental import pallas as pl
from jax.experimental.pallas import tpu as pltpu

_BM = 512

_MINV = [
    [-3.0, 27.0, 27.0, -3.0],
    [2.0, -54.0, 54.0, -2.0],
    [12.0, -12.0, -12.0, 12.0],
    [-8.0, 24.0, -24.0, 8.0],
]


def _mlp_kernel(coef_ref, x_ref, w1_ref, b1_ref, w2_ref, b2_ref, out_ref,
                k1_ref, k2_ref, k3_ref, b1p_ref):
    n = w1_ref.shape[1] // 2

    @pl.when(pl.program_id(0) == 0)
    def _premix():
        w1 = w1_ref[...]
        we = w1[:, :n]
        wo = w1[:, n:]
        k1_ref[...] = (coef_ref[0, 2] * we + coef_ref[0, 3] * wo).astype(jnp.bfloat16)
        k2_ref[...] = (coef_ref[0, 4] * we + coef_ref[0, 5] * wo).astype(jnp.bfloat16)
        k3_ref[...] = (coef_ref[0, 6] * we + coef_ref[0, 7] * wo).astype(jnp.bfloat16)
        cs = jnp.sum(w1, axis=0, keepdims=True)
        b1p_ref[...] = (b1_ref[...] + coef_ref[0, 0] * cs[:, :n]
                        + coef_ref[0, 1] * cs[:, n:])

    w = x_ref[...].astype(jnp.bfloat16) - jnp.asarray(1.5, jnp.bfloat16)
    w2 = w * w
    w3 = w2 * w
    h = jnp.dot(w, k1_ref[...], preferred_element_type=jnp.float32)
    h = h + jnp.dot(w2, k2_ref[...], preferred_element_type=jnp.float32)
    h = h + jnp.dot(w3, k3_ref[...], preferred_element_type=jnp.float32)
    h = jnp.maximum(h + b1p_ref[...], 0.0)
    out_ref[...] = jnp.dot(h, w2_ref[...],
                           preferred_element_type=jnp.float32) + b2_ref[...]


def kernel(x, table, W1, b1, W2, b2):
    B, T = x.shape
    d_hid = W1.shape[1]
    d_out = W2.shape[1]
    coef = (jnp.asarray(_MINV, jnp.float32) / 48.0) @ table
    coef = coef.reshape(1, 8)
    w1r = W1.reshape(T, 2 * d_hid)
    b1r = b1.reshape(1, d_hid)
    b2r = b2.reshape(1, d_out)
    return pl.pallas_call(
        _mlp_kernel,
        grid=(B // _BM,),
        in_specs=[
            pl.BlockSpec((1, 8), lambda i: (0, 0)),
            pl.BlockSpec((_BM, T), lambda i: (i, 0)),
            pl.BlockSpec((T, 2 * d_hid), lambda i: (0, 0)),
            pl.BlockSpec((1, d_hid), lambda i: (0, 0)),
            pl.BlockSpec((d_hid, d_out), lambda i: (0, 0)),
            pl.BlockSpec((1, d_out), lambda i: (0, 0)),
        ],
        out_specs=pl.BlockSpec((_BM, d_out), lambda i: (i, 0)),
        out_shape=jax.ShapeDtypeStruct((B, d_out), jnp.float32),
        scratch_shapes=[
            pltpu.VMEM((T, d_hid), jnp.bfloat16),
            pltpu.VMEM((T, d_hid), jnp.bfloat16),
            pltpu.VMEM((T, d_hid), jnp.bfloat16),
            pltpu.VMEM((1, d_hid), jnp.float32),
        ],
        compiler_params=pltpu.CompilerParams(
            dimension_semantics=("arbitrary",)),
    )(coef, x, w1r, b1r, W2, b2r)
